# baseline (speedup 1.0000x reference)
import jax
import jax.numpy as jnp
from jax.experimental import pallas as pl

import kernel_sc


def kernel(x, pos_emb, gamma, beta):
    b, dim, lp = x.shape
    xf = x.reshape(b * dim, lp)
    pe_t = pos_emb.T
    out = kernel_sc.sc_ln(xf, pe_t, b * dim)
    return out.reshape(b, dim, lp)


# speedup vs baseline: 2.1595x; 2.1595x over previous
import jax
import jax.numpy as jnp
from jax.experimental import pallas as pl

import kernel_sc


def _ln_kernel(x_ref, pe_ref, o_ref):
    rows, lp = x_ref.shape
    dim = pe_ref.shape[0]
    inv_d = 1.0 / dim
    pe = pe_ref[...]
    rid = jax.lax.broadcasted_iota(jnp.int32, (dim, dim), 0)
    cid = jax.lax.broadcasted_iota(jnp.int32, (dim, dim), 1)
    cmat = jnp.where(rid == cid, 1.0 - inv_d, -inv_d)
    for gi in range(rows // dim):
        sl = pl.ds(gi * dim, dim)
        v = x_ref[sl, :] + pe
        cen = jnp.dot(cmat, v, preferred_element_type=jnp.float32)
        var = jnp.sum(cen * cen, axis=0, keepdims=True) * inv_d
        o_ref[sl, :] = cen * jax.lax.rsqrt(var + 1e-5)


def kernel(x, pos_emb, gamma, beta):
    b, dim, lp = x.shape
    xf = x.reshape(b * dim, lp)
    pe_t = pos_emb.T
    tc_rows = 1920
    rows = 384
    tc_out = pl.pallas_call(
        _ln_kernel,
        grid=(tc_rows // rows,),
        in_specs=[
            pl.BlockSpec((rows, lp), lambda i: (i, 0)),
            pl.BlockSpec((dim, lp), lambda i: (0, 0)),
        ],
        out_specs=pl.BlockSpec((rows, lp), lambda i: (i, 0)),
        out_shape=jax.ShapeDtypeStruct((tc_rows, lp), x.dtype),
    )(xf[:tc_rows], pe_t)
    sc_out = kernel_sc.sc_ln(xf[tc_rows:], pe_t, b * dim - tc_rows, w=128)
    return jnp.concatenate([tc_out, sc_out], axis=0).reshape(b, dim, lp)


# speedup vs baseline: 7.0160x; 3.2489x over previous
import jax
import jax.numpy as jnp
from jax.experimental import pallas as pl
from jax.experimental.pallas import tpu as pltpu

_ROWS = 512
_DIM = 128


def _compute(src, pe, dst):
    rows, lp = src.shape
    inv_d = 1.0 / _DIM
    rid = jax.lax.broadcasted_iota(jnp.int32, (_DIM, _DIM), 0)
    cid = jax.lax.broadcasted_iota(jnp.int32, (_DIM, _DIM), 1)
    cmat = jnp.where(rid == cid, 1.0 - inv_d, -inv_d)
    for gi in range(rows // _DIM):
        sl = pl.ds(gi * _DIM, _DIM)
        v = src[sl, :] + pe
        cen = jnp.dot(cmat, v, preferred_element_type=jnp.float32)
        var = jnp.sum(cen * cen, axis=0, keepdims=True) * inv_d
        dst[sl, :] = cen * jax.lax.rsqrt(var + 1e-5)


def _ln_kernel(x_hbm, pe_ref, o_hbm, inb, outb, insem, outsem):
    n_rows = x_hbm.shape[0]
    n = n_rows // _ROWS
    pe = pe_ref[...]

    def in_cp(k, slot):
        return pltpu.make_async_copy(
            x_hbm.at[pl.ds(k * _ROWS, _ROWS), :], inb.at[slot], insem.at[slot])

    def out_cp(k, slot):
        return pltpu.make_async_copy(
            outb.at[slot], o_hbm.at[pl.ds(k * _ROWS, _ROWS), :], outsem.at[slot])

    in_cp(0, 0).start()
    in_cp(1, 1).start()
    for k in range(n):
        slot = k % 2
        in_cp(k, slot).wait()
        if k >= 2:
            out_cp(k - 2, slot).wait()
        _compute(inb.at[slot], pe, outb.at[slot])
        out_cp(k, slot).start()
        if k + 2 < n:
            in_cp(k + 2, slot).start()
    out_cp(n - 2, (n - 2) % 2).wait()
    out_cp(n - 1, (n - 1) % 2).wait()


def kernel(x, pos_emb, gamma, beta):
    b, dim, lp = x.shape
    xf = x.reshape(b * dim, lp)
    pe_t = pos_emb.T
    out = pl.pallas_call(
        _ln_kernel,
        in_specs=[
            pl.BlockSpec(memory_space=pltpu.MemorySpace.HBM),
            pl.BlockSpec((dim, lp), lambda: (0, 0)),
        ],
        out_specs=pl.BlockSpec(memory_space=pltpu.MemorySpace.HBM),
        out_shape=jax.ShapeDtypeStruct((b * dim, lp), x.dtype),
        scratch_shapes=[
            pltpu.VMEM((2, _ROWS, lp), jnp.float32),
            pltpu.VMEM((2, _ROWS, lp), jnp.float32),
            pltpu.SemaphoreType.DMA((2,)),
            pltpu.SemaphoreType.DMA((2,)),
        ],
    )(xf, pe_t)
    return out.reshape(b, dim, lp)
